# speedup vs baseline: 19.4813x; 19.4813x over previous
import functools

import jax
import jax.numpy as jnp
from jax.experimental import pallas as pl
from jax.experimental.pallas import tpu as pltpu

N = 10000
D = 128
H = 128
EMB = 128
E = 320000
UNITS = 4
EPS = 1e-7
BN_EPS = 1e-5

NPAD = 10240
BLK = 1024
KC = 2048
PW = UNITS * 4 * H


def _powers(h, exps_ref):
    xs = 1.0 / (1.0 + jnp.exp(-h))
    lg = jnp.log(xs)
    lgo = jnp.log(1.0 - xs)
    parts = []
    for j in range(UNITS):
        for col, l in ((1, lg), (3, lgo), (5, lg), (7, lgo)):
            parts.append(jnp.exp(exps_ref[j, col] * l))
    return jnp.concatenate(parts, axis=1).astype(jnp.bfloat16)


def _prep_kernel(x_ref, w_ref, b_ref, exps_ref, p_ref):
    h = jnp.dot(x_ref[...], w_ref[...], preferred_element_type=jnp.float32)
    h = h + b_ref[...]
    p_ref[...] = _powers(h, exps_ref)


def _combine(acc, laf_ref):
    out = None
    for j in range(UNITS):
        base = j * 4 * H
        s1 = acc[:, base:base + H]
        s2 = acc[:, base + H:base + 2 * H]
        s3 = acc[:, base + 2 * H:base + 3 * H]
        s4 = acc[:, base + 3 * H:base + 4 * H]
        num = (laf_ref[j, 8] * jnp.exp(laf_ref[j, 0] * jnp.log(s1 + EPS))
               + laf_ref[j, 9] * jnp.exp(laf_ref[j, 2] * jnp.log(s2 + EPS)))
        den = (laf_ref[j, 10] * jnp.exp(laf_ref[j, 4] * jnp.log(s3 + EPS))
               + laf_ref[j, 11] * jnp.exp(laf_ref[j, 6] * jnp.log(s4 + EPS)))
        term = num / (den + EPS)
        out = term if out is None else out + term
    return out * (1.0 / UNITS)


def _layer_kernel(a_ref, p_ref, laf_ref, bias_ref, gamma_ref, beta_ref,
                  w_ref, nexps_ref, out_ref, acc_ref):
    k = pl.program_id(1)

    @pl.when(k == 0)
    def _():
        acc_ref[...] = jnp.zeros_like(acc_ref)

    acc_ref[...] += jnp.dot(a_ref[...], p_ref[...],
                            preferred_element_type=jnp.float32)

    @pl.when(k == pl.num_programs(1) - 1)
    def _():
        h = _combine(acc_ref[...], laf_ref) + bias_ref[...]
        h = h * (gamma_ref[...] * (1.0 / (1.0 + BN_EPS) ** 0.5)) + beta_ref[...]
        h = jnp.maximum(h, 0.0)
        h = jnp.dot(h, w_ref[...], preferred_element_type=jnp.float32)
        out_ref[...] = _powers(h, nexps_ref)


def _final_kernel(a_ref, p_ref, laf_ref, bias_ref, out_ref, acc_ref):
    k = pl.program_id(1)

    @pl.when(k == 0)
    def _():
        acc_ref[...] = jnp.zeros_like(acc_ref)

    acc_ref[...] += jnp.dot(a_ref[...], p_ref[...],
                            preferred_element_type=jnp.float32)

    @pl.when(k == pl.num_programs(1) - 1)
    def _():
        out_ref[...] = _combine(acc_ref[...], laf_ref) + bias_ref[...]


def _prep_call(x, w, b, laf):
    grid = (NPAD // BLK,)
    return pl.pallas_call(
        _prep_kernel,
        grid=grid,
        in_specs=[
            pl.BlockSpec((BLK, D), lambda i: (i, 0)),
            pl.BlockSpec((D, H), lambda i: (0, 0)),
            pl.BlockSpec((1, H), lambda i: (0, 0)),
            pl.BlockSpec(memory_space=pltpu.SMEM),
        ],
        out_specs=pl.BlockSpec((BLK, PW), lambda i: (i, 0)),
        out_shape=jax.ShapeDtypeStruct((NPAD, PW), jnp.bfloat16),
    )(x, w, b, laf)


def _layer_call(a, p, laf, bias, gamma, beta, w, nlaf):
    grid = (NPAD // BLK, NPAD // KC)
    return pl.pallas_call(
        _layer_kernel,
        grid=grid,
        in_specs=[
            pl.BlockSpec((BLK, KC), lambda i, k: (i, k)),
            pl.BlockSpec((KC, PW), lambda i, k: (k, 0)),
            pl.BlockSpec(memory_space=pltpu.SMEM),
            pl.BlockSpec((1, H), lambda i, k: (0, 0)),
            pl.BlockSpec((1, H), lambda i, k: (0, 0)),
            pl.BlockSpec((1, H), lambda i, k: (0, 0)),
            pl.BlockSpec((H, H), lambda i, k: (0, 0)),
            pl.BlockSpec(memory_space=pltpu.SMEM),
        ],
        out_specs=pl.BlockSpec((BLK, PW), lambda i, k: (i, 0)),
        out_shape=jax.ShapeDtypeStruct((NPAD, PW), jnp.bfloat16),
        scratch_shapes=[pltpu.VMEM((BLK, PW), jnp.float32)],
        compiler_params=pltpu.CompilerParams(
            dimension_semantics=("parallel", "arbitrary")),
    )(a, p, laf, bias, gamma, beta, w, nlaf)


def _final_call(a, p, laf, bias):
    grid = (NPAD // BLK, NPAD // KC)
    return pl.pallas_call(
        _final_kernel,
        grid=grid,
        in_specs=[
            pl.BlockSpec((BLK, KC), lambda i, k: (i, k)),
            pl.BlockSpec((KC, PW), lambda i, k: (k, 0)),
            pl.BlockSpec(memory_space=pltpu.SMEM),
            pl.BlockSpec((1, EMB), lambda i, k: (0, 0)),
        ],
        out_specs=pl.BlockSpec((BLK, EMB), lambda i, k: (i, 0)),
        out_shape=jax.ShapeDtypeStruct((NPAD, EMB), jnp.float32),
        scratch_shapes=[pltpu.VMEM((BLK, PW), jnp.float32)],
        compiler_params=pltpu.CompilerParams(
            dimension_semantics=("parallel", "arbitrary")),
    )(a, p, laf, bias)


def _build_adjacency(edge_index):
    src = edge_index[0]
    dst = edge_index[1]
    a = jnp.zeros((NPAD, NPAD), jnp.float32)
    a = a.at[dst, src].add(1.0)
    return a.astype(jnp.bfloat16)


def kernel(x, edge_index, idx, W0, b0, W1, W2, laf0, laf1, laf2,
           lafbias0, lafbias1, lafbias2, gamma0, beta0, gamma1, beta1):
    a = _build_adjacency(edge_index)
    xp = jnp.pad(x, ((0, NPAD - N), (0, 0)))
    p0 = _prep_call(xp, W0, b0.reshape(1, H), laf0)
    p1 = _layer_call(a, p0, laf0, lafbias0.reshape(1, H),
                     gamma0.reshape(1, H), beta0.reshape(1, H), W1, laf1)
    p2 = _layer_call(a, p1, laf1, lafbias1.reshape(1, H),
                     gamma1.reshape(1, H), beta1.reshape(1, H), W2, laf2)
    h = _final_call(a, p2, laf2, lafbias2.reshape(1, EMB))
    return jnp.take(h[:N], idx, axis=0)
